# Initial kernel scaffold; baseline (speedup 1.0000x reference)
import functools

import jax
import jax.numpy as jnp
from jax import lax
from jax.experimental import pallas as pl
from jax.experimental.pallas import tpu as pltpu
from jax.experimental.pallas import tpu_sc as plsc

NUM_WORKERS = 32
CHUNK = 128


def _make_gather(n_chunks, d):
    mesh = plsc.VectorSubcoreMesh(core_axis_name="c", subcore_axis_name="s")

    @functools.partial(
        pl.kernel,
        out_type=jax.ShapeDtypeStruct((NUM_WORKERS, n_chunks, CHUNK, d),
                                      jnp.float32),
        mesh=mesh,
        scratch_types=[
            pltpu.VMEM((n_chunks, CHUNK), jnp.int32),
            pltpu.VMEM((CHUNK, d), jnp.float32),
            pltpu.SemaphoreType.DMA,
        ],
    )
    def gather_kernel(idx_hbm, table_hbm, out_hbm, idx_v, rows_v, sem):
        wid = lax.axis_index("s") * 2 + lax.axis_index("c")
        pltpu.sync_copy(idx_hbm.at[wid], idx_v)

        def body(j, carry):
            pltpu.async_copy(table_hbm.at[idx_v.at[j]], rows_v, sem).wait()
            pltpu.sync_copy(rows_v, out_hbm.at[wid, j])
            return carry

        lax.fori_loop(0, n_chunks, body, 0, unroll=False)

    return gather_kernel


def kernel(code, embedding_table):
    b, s = code.shape
    v, d = embedding_table.shape
    total = b * s
    assert total % (NUM_WORKERS * CHUNK) == 0
    n_chunks = total // (NUM_WORKERS * CHUNK)
    idx = code.reshape(NUM_WORKERS, n_chunks, CHUNK).astype(jnp.int32)
    out = _make_gather(n_chunks, d)(idx, embedding_table)
    return out.reshape(b, s, d)


# speedup vs baseline: 2.8862x; 2.8862x over previous
import functools

import jax
import jax.numpy as jnp
from jax import lax
from jax.experimental import pallas as pl
from jax.experimental.pallas import tpu as pltpu
from jax.experimental.pallas import tpu_sc as plsc

NUM_WORKERS = 32
CHUNK = 128


def _make_gather(n_chunks, d):
    mesh = plsc.VectorSubcoreMesh(core_axis_name="c", subcore_axis_name="s")

    @functools.partial(
        pl.kernel,
        out_type=jax.ShapeDtypeStruct((NUM_WORKERS, n_chunks, CHUNK, d),
                                      jnp.float32),
        mesh=mesh,
        scratch_types=[
            pltpu.VMEM((n_chunks, CHUNK), jnp.int32),
            pltpu.VMEM((CHUNK, d), jnp.float32),
            pltpu.SemaphoreType.DMA,
        ],
        compiler_params=pltpu.CompilerParams(use_tc_tiling_on_sc=False),
    )
    def gather_kernel(idx_hbm, table_hbm, out_hbm, idx_v, rows_v, sem):
        wid = lax.axis_index("s") * 2 + lax.axis_index("c")
        pltpu.sync_copy(idx_hbm.at[wid], idx_v)

        def body(j, carry):
            pltpu.async_copy(table_hbm.at[idx_v.at[j]], rows_v, sem).wait()
            pltpu.sync_copy(rows_v, out_hbm.at[wid, j])
            return carry

        lax.fori_loop(0, n_chunks, body, 0, unroll=False)

    return gather_kernel


def kernel(code, embedding_table):
    b, s = code.shape
    v, d = embedding_table.shape
    total = b * s
    assert total % (NUM_WORKERS * CHUNK) == 0
    n_chunks = total // (NUM_WORKERS * CHUNK)
    idx = code.reshape(NUM_WORKERS, n_chunks, CHUNK).astype(jnp.int32)
    out = _make_gather(n_chunks, d)(idx, embedding_table)
    return out.reshape(b, s, d)


# speedup vs baseline: 3.3105x; 1.1470x over previous
import functools

import jax
import jax.numpy as jnp
from jax import lax
from jax.experimental import pallas as pl
from jax.experimental.pallas import tpu as pltpu
from jax.experimental.pallas import tpu_sc as plsc

NUM_WORKERS = 32
CHUNK = 128
NBUF = 5


def _make_gather(n_chunks, d):
    mesh = plsc.VectorSubcoreMesh(core_axis_name="c", subcore_axis_name="s")

    @functools.partial(
        pl.kernel,
        out_type=jax.ShapeDtypeStruct((NUM_WORKERS, n_chunks, CHUNK, d),
                                      jnp.float32),
        mesh=mesh,
        scratch_types=(
            [pltpu.VMEM((n_chunks, CHUNK), jnp.int32),
             pltpu.VMEM((NBUF, CHUNK, d), jnp.float32)]
            + [pltpu.SemaphoreType.DMA] * (2 * NBUF)
        ),
        compiler_params=pltpu.CompilerParams(use_tc_tiling_on_sc=False),
    )
    def gather_kernel(idx_hbm, table_hbm, out_hbm, idx_v, rows_v, *sems):
        gsem = sems[:NBUF]
        wsem = sems[NBUF:]
        wid = lax.axis_index("s") * 2 + lax.axis_index("c")
        pltpu.sync_copy(idx_hbm.at[wid], idx_v)

        dummy_src = table_hbm.at[pl.ds(0, CHUNK)]

        for b in range(NBUF - 1):
            pltpu.async_copy(table_hbm.at[idx_v.at[b]], rows_v.at[b], gsem[b])

        assert n_chunks % NBUF == 0
        n_outer = n_chunks // NBUF

        def outer(g0, carry):
            for i in range(NBUF):
                j = g0 * NBUF + i
                fb = (i + NBUF - 1) % NBUF

                @pl.when(j >= 1)
                def _():
                    pltpu.make_async_copy(dummy_src, rows_v.at[fb],
                                          wsem[fb]).wait()

                @pl.when(j + NBUF - 1 < n_chunks)
                def _():
                    pltpu.async_copy(
                        table_hbm.at[idx_v.at[j + NBUF - 1]],
                        rows_v.at[fb], gsem[fb])

                pltpu.make_async_copy(dummy_src, rows_v.at[i], gsem[i]).wait()
                pltpu.async_copy(rows_v.at[i], out_hbm.at[wid, j], wsem[i])
            return carry

        lax.fori_loop(0, n_outer, outer, 0, unroll=False)
        pltpu.make_async_copy(dummy_src, rows_v.at[(n_chunks - 1) % NBUF],
                              wsem[(n_chunks - 1) % NBUF]).wait()

    return gather_kernel


def kernel(code, embedding_table):
    b, s = code.shape
    v, d = embedding_table.shape
    total = b * s
    assert total % (NUM_WORKERS * CHUNK) == 0
    n_chunks = total // (NUM_WORKERS * CHUNK)
    idx = code.reshape(NUM_WORKERS, n_chunks, CHUNK).astype(jnp.int32)
    out = _make_gather(n_chunks, d)(idx, embedding_table)
    return out.reshape(b, s, d)
